# baseline (speedup 1.0000x reference)
import functools

import jax
import jax.numpy as jnp
from jax import lax
from jax.experimental import pallas as pl
from jax.experimental.pallas import tpu as pltpu
from jax.experimental.pallas import tpu_sc as plsc

H, W = 2048, 2048
P = 64
N = 4096
PW = P + 8
CH = 16
RH = 32
NT = N // 128


def _patch_kernel(images, positions):
    info = plsc.get_sparse_core_info()
    nw = info.num_cores * info.num_subcores
    per_w = N // nw
    nchunk = per_w // CH
    nhalf = 2 * nchunk

    mesh = plsc.VectorSubcoreMesh(core_axis_name="c", subcore_axis_name="s")

    @functools.partial(
        pl.kernel,
        mesh=mesh,
        compiler_params=pltpu.CompilerParams(
            use_tc_tiling_on_sc=False, needs_layout_passes=False
        ),
        out_type=jax.ShapeDtypeStruct((P, 8, NT * 8, 128), jnp.float32),
        scratch_types=[
            pltpu.VMEM((2 * per_w,), jnp.int32),
            pltpu.SMEM((per_w, 2), jnp.int32),
            pltpu.VMEM((2, CH, RH + 1, PW + 1), jnp.float32),
            pltpu.VMEM((RH, 8, 8, CH), jnp.float32),
            pltpu.SemaphoreType.DMA((2,)),
            pltpu.SemaphoreType.DMA,
        ],
    )
    def k(img_hbm, pos_hbm, out_hbm, pos_v, pos_s, ibuf, tbuf, in_sem,
          out_sem):
        wid = lax.axis_index("s") * info.num_cores + lax.axis_index("c")
        base = wid * per_w
        pltpu.sync_copy(pos_hbm.at[pl.ds(2 * base, 2 * per_w)], pos_v)
        for j in range(per_w // 8):
            v = pos_v[pl.ds(16 * j, 16)]
            for t in range(8):
                pos_s[8 * j + t, 0] = v[2 * t]
                pos_s[8 * j + t, 1] = v[2 * t + 1]

        lane = lax.iota(jnp.int32, 16)

        def start_in(h, slot):
            cbase = (h // 2) * CH
            roff = (h % 2) * RH
            for t in range(CH):
                r = pos_s[cbase + t, 0]
                c = pos_s[cbase + t, 1]
                c8 = pl.multiple_of((c // 8) * 8, 8)
                pltpu.make_async_copy(
                    img_hbm.at[pl.ds(r + roff, RH), pl.ds(c8, PW)],
                    ibuf.at[slot, t, pl.ds(0, RH), pl.ds(0, PW)],
                    in_sem.at[slot],
                ).start()

        def wait_in(slot):
            for t in range(CH):
                pltpu.make_async_copy(
                    img_hbm.at[pl.ds(0, RH), pl.ds(0, PW)],
                    ibuf.at[slot, t, pl.ds(0, RH), pl.ds(0, PW)],
                    in_sem.at[slot],
                ).wait()

        def out_dst(h):
            cbase = (h // 2) * CH
            roff = (h % 2) * RH
            return out_hbm.at[
                pl.ds(roff, RH),
                slice(None),
                pl.ds(wid * 8, 8),
                pl.ds(cbase, CH),
            ]

        def start_out(h):
            pltpu.make_async_copy(tbuf, out_dst(h), out_sem).start()

        def wait_out(h):
            pltpu.make_async_copy(tbuf, out_dst(h), out_sem).wait()

        def shuffle(h, slot):
            cbase = (h // 2) * CH
            src = ibuf.at[slot]
            cvals = plsc.load_gather(pos_v, [2 * (cbase + lane) + 1])
            svec = lax.rem(cvals, 8)
            svjr = [svec + jr for jr in range(8)]

            def row_body(row, _):
                drow = jnp.full((16,), row, dtype=jnp.int32)
                for jt in range(8):
                    for jr in range(8):
                        v = plsc.load_gather(
                            src, [lane, drow, svjr[jr] + jt * 8]
                        )
                        tbuf[row, jt, jr, :] = v
                return 0

            lax.fori_loop(0, RH, row_body, 0)

        start_in(0, 0)

        def body(h, _):
            slot = lax.rem(h, 2)
            nslot = lax.rem(h + 1, 2)

            @pl.when(h + 1 < nhalf)
            def _():
                start_in(h + 1, nslot)

            wait_in(slot)

            @pl.when(h >= 1)
            def _():
                wait_out(h - 1)

            shuffle(h, slot)
            start_out(h)
            return 0

        lax.fori_loop(0, nhalf, body, 0)
        wait_out(nhalf - 1)

    return k(images, positions.reshape(-1))


def kernel(images, positions, widths):
    del widths
    out5 = _patch_kernel(images, positions).reshape(P, 8, NT, 8, 128)
    return out5.transpose(2, 4, 0, 1, 3).reshape(N, P, P)


# speedup vs baseline: 1.1553x; 1.1553x over previous
import functools

import jax
import jax.numpy as jnp
from jax import lax
from jax.experimental import pallas as pl
from jax.experimental.pallas import tpu as pltpu
from jax.experimental.pallas import tpu_sc as plsc

H, W = 2048, 2048
P = 64
N = 4096
PW = P + 8
CH = 16
RH = 32
NT = N // 128


def _patch_kernel(images, positions):
    info = plsc.get_sparse_core_info()
    nw = info.num_cores * info.num_subcores
    per_w = N // nw
    nchunk = per_w // CH
    nhalf = 2 * nchunk

    mesh = plsc.VectorSubcoreMesh(core_axis_name="c", subcore_axis_name="s")

    @functools.partial(
        pl.kernel,
        mesh=mesh,
        compiler_params=pltpu.CompilerParams(
            use_tc_tiling_on_sc=False, needs_layout_passes=False
        ),
        out_type=jax.ShapeDtypeStruct((P, 8, NT, 8, 128), jnp.float32),
        scratch_types=[
            pltpu.VMEM((2 * per_w,), jnp.int32),
            pltpu.SMEM((per_w, 2), jnp.int32),
            pltpu.VMEM((2, CH, RH, PW), jnp.float32),
            pltpu.VMEM((RH, 8, 1, 8, CH), jnp.float32),
            pltpu.SemaphoreType.DMA((2,)),
            pltpu.SemaphoreType.DMA,
        ],
    )
    def k(img_hbm, pos_hbm, out_hbm, pos_v, pos_s, ibuf, tbuf, in_sem,
          out_sem):
        wid = lax.axis_index("s") * info.num_cores + lax.axis_index("c")
        base = wid * per_w
        pltpu.sync_copy(pos_hbm.at[pl.ds(2 * base, 2 * per_w)], pos_v)
        for j in range(per_w // 8):
            v = pos_v[pl.ds(16 * j, 16)]
            for t in range(8):
                pos_s[8 * j + t, 0] = v[2 * t]
                pos_s[8 * j + t, 1] = v[2 * t + 1]

        lane = lax.iota(jnp.int32, 16)
        jt_idx = [(kk * 16 + lane) // 8 for kk in range(P // 16)]
        jr_idx = lane % 8
        zero_idx = jnp.zeros((16,), dtype=jnp.int32)

        def start_in(h, slot):
            cbase = (h // 2) * CH
            roff = (h % 2) * RH
            for t in range(CH):
                r = pos_s[cbase + t, 0]
                c = pos_s[cbase + t, 1]
                c8 = pl.multiple_of((c // 8) * 8, 8)
                pltpu.make_async_copy(
                    img_hbm.at[pl.ds(r + roff, RH), pl.ds(c8, PW)],
                    ibuf.at[slot, t],
                    in_sem.at[slot],
                ).start()

        def wait_in(slot):
            for t in range(CH):
                pltpu.make_async_copy(
                    img_hbm.at[pl.ds(0, RH), pl.ds(0, PW)],
                    ibuf.at[slot, t],
                    in_sem.at[slot],
                ).wait()

        def out_dst(h):
            cbase = (h // 2) * CH
            roff = (h % 2) * RH
            return out_hbm.at[
                pl.ds(roff, RH),
                slice(None),
                pl.ds(wid, 1),
                slice(None),
                pl.ds(cbase, CH),
            ]

        def start_out(h):
            pltpu.make_async_copy(tbuf, out_dst(h), out_sem).start()

        def wait_out(h):
            pltpu.make_async_copy(tbuf, out_dst(h), out_sem).wait()

        def shuffle(h, slot):
            cbase = (h // 2) * CH

            def patch_body(t, _):
                s = pos_s[cbase + t, 1] % 8
                src = ibuf.at[slot, t]
                pn = jnp.full((16,), t, dtype=jnp.int32)
                cidx = [s + kk * 16 + lane for kk in range(P // 16)]

                def row_body(row, _):
                    ridx = jnp.full((16,), row, dtype=jnp.int32)
                    for kk in range(P // 16):
                        v = plsc.load_gather(src, [ridx, cidx[kk]])
                        plsc.store_scatter(
                            tbuf,
                            [ridx, jt_idx[kk], zero_idx, jr_idx, pn],
                            v,
                        )
                    return 0

                lax.fori_loop(0, RH, row_body, 0)
                return 0

            lax.fori_loop(0, CH, patch_body, 0)

        start_in(0, 0)

        def body(h, _):
            slot = lax.rem(h, 2)
            nslot = lax.rem(h + 1, 2)

            @pl.when(h + 1 < nhalf)
            def _():
                start_in(h + 1, nslot)

            wait_in(slot)

            @pl.when(h >= 1)
            def _():
                wait_out(h - 1)

            shuffle(h, slot)
            start_out(h)
            return 0

        lax.fori_loop(0, nhalf, body, 0)
        wait_out(nhalf - 1)

    return k(images, positions.reshape(-1))


def kernel(images, positions, widths):
    del widths
    out5 = _patch_kernel(images, positions)
    return out5.transpose(2, 4, 0, 1, 3).reshape(N, P, P)


# speedup vs baseline: 1.1777x; 1.0194x over previous
import functools

import jax
import jax.numpy as jnp
from jax import lax
from jax.experimental import pallas as pl
from jax.experimental.pallas import tpu as pltpu
from jax.experimental.pallas import tpu_sc as plsc

H, W = 2048, 2048
P = 64
N = 4096
PW = P + 8
CH = 16
RH = 16
NT = N // 128


def _patch_kernel(images, positions):
    info = plsc.get_sparse_core_info()
    nw = info.num_cores * info.num_subcores
    per_w = N // nw
    nchunk = per_w // CH
    nhalf = 4 * nchunk

    mesh = plsc.VectorSubcoreMesh(core_axis_name="c", subcore_axis_name="s")

    @functools.partial(
        pl.kernel,
        mesh=mesh,
        compiler_params=pltpu.CompilerParams(
            use_tc_tiling_on_sc=False, needs_layout_passes=False
        ),
        out_type=jax.ShapeDtypeStruct((P, 8, NT, 8, 128), jnp.float32),
        scratch_types=[
            pltpu.VMEM((2 * per_w,), jnp.int32),
            pltpu.SMEM((per_w, 2), jnp.int32),
            pltpu.VMEM((2, CH, RH, PW), jnp.float32),
            pltpu.VMEM((2, RH, 8, 1, 8, CH), jnp.float32),
            pltpu.SemaphoreType.DMA((2,)),
            pltpu.SemaphoreType.DMA((2,)),
        ],
    )
    def k(img_hbm, pos_hbm, out_hbm, pos_v, pos_s, ibuf, tbuf, in_sem,
          out_sem):
        wid = lax.axis_index("s") * info.num_cores + lax.axis_index("c")
        base = wid * per_w
        pltpu.sync_copy(pos_hbm.at[pl.ds(2 * base, 2 * per_w)], pos_v)
        for j in range(per_w // 8):
            v = pos_v[pl.ds(16 * j, 16)]
            for t in range(8):
                pos_s[8 * j + t, 0] = v[2 * t]
                pos_s[8 * j + t, 1] = v[2 * t + 1]

        lane = lax.iota(jnp.int32, 16)
        jt_idx = [(kk * 16 + lane) // 8 for kk in range(P // 16)]
        jr_idx = lane % 8
        zero_idx = jnp.zeros((16,), dtype=jnp.int32)

        def start_in(h, slot):
            cbase = (h // 4) * CH
            roff = (h % 4) * RH
            for t in range(CH):
                r = pos_s[cbase + t, 0]
                c = pos_s[cbase + t, 1]
                c8 = pl.multiple_of((c // 8) * 8, 8)
                pltpu.make_async_copy(
                    img_hbm.at[pl.ds(r + roff, RH), pl.ds(c8, PW)],
                    ibuf.at[slot, t],
                    in_sem.at[slot],
                ).start()

        def wait_in(slot):
            for t in range(CH):
                pltpu.make_async_copy(
                    img_hbm.at[pl.ds(0, RH), pl.ds(0, PW)],
                    ibuf.at[slot, t],
                    in_sem.at[slot],
                ).wait()

        def out_dst(h):
            cbase = (h // 4) * CH
            roff = (h % 4) * RH
            return out_hbm.at[
                pl.ds(roff, RH),
                slice(None),
                pl.ds(wid, 1),
                slice(None),
                pl.ds(cbase, CH),
            ]

        def start_out(h, ts):
            pltpu.make_async_copy(tbuf.at[ts], out_dst(h), out_sem.at[ts]).start()

        def wait_out(h, ts):
            pltpu.make_async_copy(tbuf.at[ts], out_dst(h), out_sem.at[ts]).wait()

        def shuffle(h, slot, ts):
            cbase = (h // 4) * CH
            tdst = tbuf.at[ts]

            def patch_body(t, _):
                s = pos_s[cbase + t, 1] % 8
                src = ibuf.at[slot, t]
                pn = jnp.full((16,), t, dtype=jnp.int32)
                cidx = [s + kk * 16 + lane for kk in range(P // 16)]

                def row_body(row, _):
                    ridx = jnp.full((16,), row, dtype=jnp.int32)
                    for kk in range(P // 16):
                        v = plsc.load_gather(src, [ridx, cidx[kk]])
                        plsc.store_scatter(
                            tdst,
                            [ridx, jt_idx[kk], zero_idx, jr_idx, pn],
                            v,
                        )
                    return 0

                lax.fori_loop(0, RH, row_body, 0)
                return 0

            lax.fori_loop(0, CH, patch_body, 0)

        start_in(0, 0)

        def body(h, _):
            slot = lax.rem(h, 2)
            nslot = lax.rem(h + 1, 2)

            @pl.when(h + 1 < nhalf)
            def _():
                start_in(h + 1, nslot)

            wait_in(slot)
            ts = lax.rem(h, 2)

            @pl.when(h >= 2)
            def _():
                wait_out(h - 2, ts)

            shuffle(h, slot, ts)
            start_out(h, ts)
            return 0

        lax.fori_loop(0, nhalf, body, 0)
        wait_out(nhalf - 2, lax.rem(nhalf - 2, 2))
        wait_out(nhalf - 1, lax.rem(nhalf - 1, 2))

    return k(images, positions.reshape(-1))


def kernel(images, positions, widths):
    del widths
    out5 = _patch_kernel(images, positions)
    return out5.transpose(2, 4, 0, 1, 3).reshape(N, P, P)


# speedup vs baseline: 1.1778x; 1.0001x over previous
import functools

import jax
import jax.numpy as jnp
from jax import lax
from jax.experimental import pallas as pl
from jax.experimental.pallas import tpu as pltpu
from jax.experimental.pallas import tpu_sc as plsc

H, W = 2048, 2048
P = 64
N = 4096
PW = P + 8
CH = 16
RH = 16
NT = N // 128


def _patch_kernel(images, positions):
    info = plsc.get_sparse_core_info()
    nw = info.num_cores * info.num_subcores
    per_w = N // nw
    nchunk = per_w // CH
    ngrp = 4 * nchunk

    mesh = plsc.VectorSubcoreMesh(core_axis_name="c", subcore_axis_name="s")

    @functools.partial(
        pl.kernel,
        mesh=mesh,
        compiler_params=pltpu.CompilerParams(
            use_tc_tiling_on_sc=False, needs_layout_passes=False
        ),
        out_type=jax.ShapeDtypeStruct((P, 8, NT, 8, 128), jnp.float32),
        scratch_types=[
            pltpu.VMEM((2 * per_w,), jnp.int32),
            pltpu.SMEM((per_w, 2), jnp.int32),
            pltpu.VMEM((2, CH, RH, PW), jnp.float32),
            pltpu.VMEM((2, RH, 8, 1, 8, CH), jnp.float32),
            pltpu.SemaphoreType.DMA((2,)),
            pltpu.SemaphoreType.DMA((2,)),
        ],
    )
    def k(img_hbm, pos_hbm, out_hbm, pos_v, pos_s, ibuf, tbuf, in_sem,
          out_sem):
        wid = lax.axis_index("s") * info.num_cores + lax.axis_index("c")
        base = wid * per_w
        pltpu.sync_copy(pos_hbm.at[pl.ds(2 * base, 2 * per_w)], pos_v)
        for j in range(per_w // 8):
            v = pos_v[pl.ds(16 * j, 16)]
            for t in range(8):
                pos_s[8 * j + t, 0] = v[2 * t]
                pos_s[8 * j + t, 1] = v[2 * t + 1]

        lane = lax.iota(jnp.int32, 16)
        jt_idx = [(kk * 16 + lane) // 8 for kk in range(P // 16)]
        jr_idx = lane % 8
        zero_idx = jnp.zeros((16,), dtype=jnp.int32)

        def start_in(h, slot):
            cbase = (h // 4) * CH
            roff = (h % 4) * RH
            for t in range(CH):
                r = pos_s[cbase + t, 0]
                c = pos_s[cbase + t, 1]
                c8 = pl.multiple_of((c // 8) * 8, 8)
                pltpu.make_async_copy(
                    img_hbm.at[pl.ds(r + roff, RH), pl.ds(c8, PW)],
                    ibuf.at[slot, t],
                    in_sem.at[slot],
                ).start()

        def wait_in(slot):
            for t in range(CH):
                pltpu.make_async_copy(
                    img_hbm.at[pl.ds(0, RH), pl.ds(0, PW)],
                    ibuf.at[slot, t],
                    in_sem.at[slot],
                ).wait()

        def out_dst(h):
            cbase = (h // 4) * CH
            roff = (h % 4) * RH
            return out_hbm.at[
                pl.ds(roff, RH),
                slice(None),
                pl.ds(wid, 1),
                slice(None),
                pl.ds(cbase, CH),
            ]

        def start_out(h, ts):
            pltpu.make_async_copy(tbuf.at[ts], out_dst(h), out_sem.at[ts]).start()

        def wait_out(h, ts):
            pltpu.make_async_copy(tbuf.at[ts], out_dst(h), out_sem.at[ts]).wait()

        def shuffle(h, slot, ts):
            cbase = (h // 4) * CH
            tdst = tbuf.at[ts]

            def patch_body(t, _):
                s = pos_s[cbase + t, 1] % 8
                src = ibuf.at[slot, t]
                pn = jnp.full((16,), t, dtype=jnp.int32)
                cidx = [s + kk * 16 + lane for kk in range(P // 16)]

                def row_body(row, _):
                    ridx = jnp.full((16,), row, dtype=jnp.int32)
                    for kk in range(P // 16):
                        v = plsc.load_gather(src, [ridx, cidx[kk]])
                        plsc.store_scatter(
                            tdst,
                            [ridx, jt_idx[kk], zero_idx, jr_idx, pn],
                            v,
                        )
                    return 0

                lax.fori_loop(0, RH, row_body, 0)
                return 0

            lax.fori_loop(0, CH, patch_body, 0)

        start_in(0, 0)

        def body(h, _):
            slot = lax.rem(h, 2)
            nslot = lax.rem(h + 1, 2)

            @pl.when(h + 1 < ngrp)
            def _():
                start_in(h + 1, nslot)

            wait_in(slot)
            ts = lax.rem(h, 2)

            @pl.when(h >= 2)
            def _():
                wait_out(h - 2, ts)

            shuffle(h, slot, ts)
            start_out(h, ts)
            return 0

        lax.fori_loop(0, ngrp, body, 0)
        wait_out(ngrp - 2, lax.rem(ngrp - 2, 2))
        wait_out(ngrp - 1, lax.rem(ngrp - 1, 2))

    return k(images, positions.reshape(-1))


def kernel(images, positions, widths):
    del widths
    out5 = _patch_kernel(images, positions)
    return out5.transpose(2, 4, 0, 1, 3).reshape(N, P, P)
